# baseline (speedup 1.0000x reference)
import functools

import jax
import jax.numpy as jnp
from jax import lax
from jax.experimental import pallas as pl
from jax.experimental.pallas import tpu as pltpu
from jax.experimental.pallas import tpu_sc as plsc

CODEBOOK_SIZE = 8192
CODEBOOK_DIM = 256
N_TOKENS = 262144

NUM_CORES = 2
NUM_SUBCORES = 16
NUM_WORKERS = NUM_CORES * NUM_SUBCORES
B_PER_W = N_TOKENS // NUM_WORKERS
CHUNK = 64
NCHUNK = B_PER_W // CHUNK
NBUF = 3

_MESH = plsc.VectorSubcoreMesh(core_axis_name="c", subcore_axis_name="s")


@functools.partial(
    pl.kernel,
    mesh=_MESH,
    out_type=jax.ShapeDtypeStruct((N_TOKENS, 2, 128), jnp.float32),
    scratch_types=[
        pltpu.VMEM((NCHUNK, CHUNK), jnp.int32),
        pltpu.VMEM((NBUF, CHUNK, 2, 128), jnp.float32),
        pltpu.SemaphoreType.DMA,
        pltpu.SemaphoreType.DMA,
        pltpu.SemaphoreType.DMA,
        pltpu.SemaphoreType.DMA,
        pltpu.SemaphoreType.DMA,
        pltpu.SemaphoreType.DMA,
    ],
)
def _codebook_gather(weight_hbm, idx_hbm, out_hbm, idx_v, rows_v,
                     gsem0, gsem1, gsem2, ssem0, ssem1, ssem2):
    wid = lax.axis_index("s") * NUM_CORES + lax.axis_index("c")
    base = wid * B_PER_W
    gsems = [gsem0, gsem1, gsem2]
    ssems = [ssem0, ssem1, ssem2]

    pltpu.sync_copy(idx_hbm.at[wid], idx_v)

    def start_gather(g, buf):
        pltpu.make_async_copy(
            weight_hbm.at[idx_v.at[g]], rows_v.at[buf], gsems[buf]).start()

    def wait_gather(buf):
        pltpu.make_async_copy(
            weight_hbm.at[idx_v.at[0]], rows_v.at[buf], gsems[buf]).wait()

    def start_store(g, buf):
        pltpu.make_async_copy(
            rows_v.at[buf], out_hbm.at[pl.ds(base + g * CHUNK, CHUNK)],
            ssems[buf]).start()

    def wait_store(buf):
        pltpu.make_async_copy(
            rows_v.at[buf], out_hbm.at[pl.ds(base, CHUNK)], ssems[buf]).wait()

    start_gather(0, 0)
    start_gather(1, 1)
    start_gather(2, 2)
    wait_gather(0)
    start_store(0, 0)
    wait_store(0)
    start_gather(3, 0)
    wait_gather(1)
    start_store(1, 1)

    def steady(i, carry):
        for j in range(NBUF):
            g = 2 + NBUF * i + j
            cur = (2 + j) % NBUF
            prv = (1 + j) % NBUF
            wait_store(prv)
            start_gather(g + 2, prv)
            wait_gather(cur)
            start_store(g, cur)
        return carry

    lax.fori_loop(0, (NCHUNK - 5) // NBUF, steady, 0)

    g = NCHUNK - 3
    wait_store((g - 1) % NBUF)
    start_gather(NCHUNK - 1, (g - 1) % NBUF)
    wait_gather(g % NBUF)
    start_store(g, g % NBUF)
    for g in (NCHUNK - 2, NCHUNK - 1):
        wait_gather(g % NBUF)
        start_store(g, g % NBUF)
    for b in range(NBUF):
        wait_store(b)


def kernel(embed_id, weight):
    idx = embed_id.astype(jnp.int32).reshape(NUM_WORKERS, NCHUNK, CHUNK)
    w3 = weight.reshape(CODEBOOK_SIZE, 2, 128)
    return _codebook_gather(w3, idx).reshape(N_TOKENS, CODEBOOK_DIM)
